# baseline (speedup 1.0000x reference)
import jax
import jax.numpy as jnp
from jax.experimental import pallas as pl
from jax.experimental.pallas import tpu as pltpu


_NT = 2
_TL = 1024


def _softmax_manual(x_hbm, o_hbm, bufs, rsems, wsems):
    core = pl.program_id(0)
    base = core * (_NT * _TL)

    for j in range(_NT):
        pltpu.make_async_copy(
            x_hbm.at[:, pl.ds(base + j * _TL, _TL)],
            bufs.at[j], rsems.at[j]).start()

    for j in range(_NT):
        dst = bufs.at[j]
        pltpu.make_async_copy(dst, dst, rsems.at[j]).wait()
        m = jnp.max(dst[...], axis=0, keepdims=True)
        dst[...] = jnp.exp(dst[...] - m)
        s = jnp.sum(dst[...], axis=0, keepdims=True)
        dst[...] = dst[...] * (1.0 / s)
        pltpu.make_async_copy(
            dst, o_hbm.at[:, pl.ds(base + j * _TL, _TL)], wsems.at[j]).start()

    for j in range(_NT):
        pltpu.make_async_copy(
            bufs.at[j], o_hbm.at[:, pl.ds(base + j * _TL, _TL)],
            wsems.at[j]).wait()


def kernel(x):
    n, d = x.shape
    return pl.pallas_call(
        _softmax_manual,
        out_shape=jax.ShapeDtypeStruct((n, d), x.dtype),
        grid=(2,),
        in_specs=[pl.BlockSpec(memory_space=pl.ANY)],
        out_specs=pl.BlockSpec(memory_space=pl.ANY),
        scratch_shapes=[
            pltpu.VMEM((_NT, n, _TL), jnp.float32),
            pltpu.SemaphoreType.DMA((_NT,)),
            pltpu.SemaphoreType.DMA((_NT,)),
        ],
        compiler_params=pltpu.CompilerParams(
            dimension_semantics=("parallel",),
            vmem_limit_bytes=56 * 1024 * 1024,
        ),
    )(x)


# speedup vs baseline: 1.0839x; 1.0839x over previous
import jax
import jax.numpy as jnp
from jax.experimental import pallas as pl
from jax.experimental.pallas import tpu as pltpu


_TILES = (512, 768, 512, 256)


def _softmax_manual(x_hbm, o_hbm, b0, b1, b2, b3, rsems, wsems):
    bufs = (b0, b1, b2, b3)
    core = pl.program_id(0)
    base = core * sum(_TILES)
    offs = []
    off = 0
    for tl in _TILES:
        offs.append(off)
        off += tl

    for j, (tl, o) in enumerate(zip(_TILES, offs)):
        pltpu.make_async_copy(
            x_hbm.at[:, pl.ds(base + o, tl)], bufs[j], rsems.at[j]).start()

    for j, (tl, o) in enumerate(zip(_TILES, offs)):
        dst = bufs[j]
        pltpu.make_async_copy(dst, dst, rsems.at[j]).wait()
        m = jnp.max(dst[...], axis=0, keepdims=True)
        dst[...] = jnp.exp(dst[...] - m)
        s = jnp.sum(dst[...], axis=0, keepdims=True)
        dst[...] = dst[...] * (1.0 / s)
        pltpu.make_async_copy(
            dst, o_hbm.at[:, pl.ds(base + o, tl)], wsems.at[j]).start()

    for j, (tl, o) in enumerate(zip(_TILES, offs)):
        pltpu.make_async_copy(
            bufs[j], o_hbm.at[:, pl.ds(base + o, tl)], wsems.at[j]).wait()


def kernel(x):
    n, d = x.shape
    return pl.pallas_call(
        _softmax_manual,
        out_shape=jax.ShapeDtypeStruct((n, d), x.dtype),
        grid=(2,),
        in_specs=[pl.BlockSpec(memory_space=pl.ANY)],
        out_specs=pl.BlockSpec(memory_space=pl.ANY),
        scratch_shapes=[pltpu.VMEM((n, tl), jnp.float32) for tl in _TILES] + [
            pltpu.SemaphoreType.DMA((len(_TILES),)),
            pltpu.SemaphoreType.DMA((len(_TILES),)),
        ],
        compiler_params=pltpu.CompilerParams(
            dimension_semantics=("parallel",),
            vmem_limit_bytes=56 * 1024 * 1024,
        ),
    )(x)
